# Initial kernel scaffold; baseline (speedup 1.0000x reference)
import jax
import jax.numpy as jnp
from jax.experimental import pallas as pl


def kernel(ref_knn_masks, src_knn_masks, matching_score_map, node_corr_scores):
    raise NotImplementedError("write your pallas kernel here")


# speedup vs baseline: 26.0655x; 26.0655x over previous
import jax
import jax.numpy as jnp
from jax import lax
from jax.experimental import pallas as pl
from jax.experimental.pallas import tpu as pltpu

P, R, S = 1024, 128, 128
PB = 8


def _top3_sel(w, axis):
    iota = lax.broadcasted_iota(jnp.int32, w.shape, axis)
    sel = jnp.zeros(w.shape, jnp.bool_)
    neg = jnp.float32(-jnp.inf)
    for _ in range(3):
        m = jnp.max(w, axis=axis, keepdims=True)
        eq = w == m
        idx = jnp.min(jnp.where(eq, iota, jnp.int32(w.shape[axis])), axis=axis,
                      keepdims=True)
        pick = iota == idx
        sel = jnp.logical_or(sel, pick)
        w = jnp.where(pick, neg, w)
    return sel


def _body(mr_ref, ms_ref, x_ref, score_ref, corr_ref):
    x = x_ref[...]
    sel_r = _top3_sel(x, 2)
    sel_c = _top3_sel(x, 1)
    a = jnp.exp(x)
    cnt = sel_r.astype(jnp.float32) + sel_c.astype(jnp.float32)
    score_ref[...] = a * (cnt * jnp.float32(0.5))
    mr = mr_ref[...].astype(jnp.float32)
    ms = ms_ref[...].astype(jnp.float32)
    mask = (mr[:, :, None] * ms[:, None, :]) > 0
    corr = jnp.logical_and(jnp.logical_and(jnp.logical_or(sel_r, sel_c), a > 0),
                           mask)
    corr_ref[...] = corr


def kernel(ref_knn_masks, src_knn_masks, matching_score_map, node_corr_scores):
    del node_corr_scores
    grid = (P // PB,)
    score, corr = pl.pallas_call(
        _body,
        grid=grid,
        in_specs=[
            pl.BlockSpec((PB, R), lambda i: (i, 0)),
            pl.BlockSpec((PB, S), lambda i: (i, 0)),
            pl.BlockSpec((PB, R, S), lambda i: (i, 0, 0)),
        ],
        out_specs=[
            pl.BlockSpec((PB, R, S), lambda i: (i, 0, 0)),
            pl.BlockSpec((PB, R, S), lambda i: (i, 0, 0)),
        ],
        out_shape=[
            jax.ShapeDtypeStruct((P, R, S), jnp.float32),
            jax.ShapeDtypeStruct((P, R, S), jnp.bool_),
        ],
    )(ref_knn_masks, src_knn_masks, matching_score_map)
    return score, corr


# speedup vs baseline: 33.8630x; 1.2991x over previous
import jax
import jax.numpy as jnp
from jax import lax
from jax.experimental import pallas as pl
from jax.experimental.pallas import tpu as pltpu

P, R, S = 1024, 128, 128
PB = 8


def _top3_sel(w, axis):
    iota = lax.broadcasted_iota(jnp.int32, w.shape, axis).astype(jnp.float32)
    neg = jnp.float32(-jnp.inf)
    big = jnp.float32(1e9)
    for _ in range(3):
        m = jnp.max(w, axis=axis, keepdims=True)
        eq = w == m
        idx = jnp.min(jnp.where(eq, iota, big), axis=axis, keepdims=True)
        w = jnp.where(iota == idx, neg, w)
    return w == neg


def _body(mr_ref, ms_ref, x_ref, score_ref, corr_ref):
    x = x_ref[...]
    sel_r = _top3_sel(x, 2)
    sel_c = _top3_sel(x, 1)
    a = jnp.exp(x)
    half = jnp.float32(0.5)
    zero = jnp.float32(0.0)
    cnt = jnp.where(sel_r, half, zero) + jnp.where(sel_c, half, zero)
    score = a * cnt
    score_ref[...] = score
    mr = mr_ref[...].astype(jnp.float32)
    ms = ms_ref[...].astype(jnp.float32)
    mask = (mr[:, :, None] * ms[:, None, :]) > 0
    corr_ref[...] = jnp.logical_and(score > zero, mask)


def kernel(ref_knn_masks, src_knn_masks, matching_score_map, node_corr_scores):
    del node_corr_scores
    grid = (P // PB,)
    score, corr = pl.pallas_call(
        _body,
        grid=grid,
        in_specs=[
            pl.BlockSpec((PB, R), lambda i: (i, 0)),
            pl.BlockSpec((PB, S), lambda i: (i, 0)),
            pl.BlockSpec((PB, R, S), lambda i: (i, 0, 0)),
        ],
        out_specs=[
            pl.BlockSpec((PB, R, S), lambda i: (i, 0, 0)),
            pl.BlockSpec((PB, R, S), lambda i: (i, 0, 0)),
        ],
        out_shape=[
            jax.ShapeDtypeStruct((P, R, S), jnp.float32),
            jax.ShapeDtypeStruct((P, R, S), jnp.bool_),
        ],
    )(ref_knn_masks, src_knn_masks, matching_score_map)
    return score, corr
